# baseline (speedup 1.0000x reference)
import functools

import jax
import jax.numpy as jnp
from jax import lax
from jax.experimental import pallas as pl
from jax.experimental.pallas import tpu as pltpu
from jax.experimental.pallas import tpu_sc as plsc

_B = 4096
_L = 200
_H = 64
_LANES = 16
_NC = 2
_NS = 16
_NW = _NC * _NS

_CHUNK = 128
_CPW = (_B * _L) // (_CHUNK * _NW)
_NBUF = 4
_LOOKAHEAD = 2
_EPS = 1e-12


def _rsqrt_vec(x):
    i = plsc.bitcast(x, jnp.int32)
    i = jnp.int32(0x5F3759DF) - lax.shift_right_arithmetic(i, 1)
    y = plsc.bitcast(i, jnp.float32)
    for _ in range(3):
        y = y * (1.5 - 0.5 * x * y * y)
    return y


def _sc_body(ids_hbm, table_hbm, pos_hbm, gamma_hbm, beta_hbm, out_hbm,
             idx_all, pos_v, gb_v, bufs, gsem, wsem):
    w = lax.axis_index("s") * _NC + lax.axis_index("c")

    pltpu.sync_copy(ids_hbm.at[pl.ds(w * _CPW, _CPW)], idx_all)
    pltpu.sync_copy(pos_hbm, pos_v)
    pltpu.sync_copy(gamma_hbm, gb_v.at[0])
    pltpu.sync_copy(beta_hbm, gb_v.at[1])

    gvecs = [gb_v[0, pl.ds(k * _LANES, _LANES)] for k in range(_H // _LANES)]
    bvecs = [gb_v[1, pl.ds(k * _LANES, _LANES)] for k in range(_H // _LANES)]

    iot = lax.iota(jnp.int32, _LANES)
    perms = [iot ^ d for d in (1, 2, 4, 8)]

    def allsum(x):
        for p in perms:
            x = x + x.at[p].get(mode="promise_in_bounds")
        return x

    for j in range(_LOOKAHEAD):
        pltpu.async_copy(table_hbm.at[idx_all.at[j]], bufs.at[j], gsem.at[j])

    def chunk_body(i, carry):
        b = lax.rem(i, _NBUF)
        g = w * _CPW + i
        pltpu.make_async_copy(
            table_hbm.at[idx_all.at[i]], bufs.at[b], gsem.at[b]).wait()

        lbase = lax.rem(g * _CHUNK, _L)

        def row_body(r, c):
            l0 = lbase + r
            l = jnp.where(l0 >= _L, l0 - _L, l0)
            xs = []
            for k in range(_H // _LANES):
                xs.append(bufs[b, r, pl.ds(k * _LANES, _LANES)]
                          + pos_v[l, pl.ds(k * _LANES, _LANES)])
            s = (xs[0] + xs[1]) + (xs[2] + xs[3])
            sq = ((xs[0] * xs[0] + xs[1] * xs[1])
                  + (xs[2] * xs[2] + xs[3] * xs[3]))
            mean_v = allsum(s) * (1.0 / _H)
            ex2_v = allsum(sq) * (1.0 / _H)
            var_v = jnp.maximum(ex2_v - mean_v * mean_v, 0.0)
            rstd = _rsqrt_vec(var_v + _EPS)
            for k in range(_H // _LANES):
                y = (xs[k] - mean_v) * (rstd * gvecs[k]) + bvecs[k]
                bufs[b, r, pl.ds(k * _LANES, _LANES)] = y
            return c

        lax.fori_loop(0, _CHUNK, row_body, 0)

        pltpu.async_copy(
            bufs.at[b], out_hbm.at[pl.ds(g * _CHUNK, _CHUNK)], wsem.at[b])

        @pl.when(i + _LOOKAHEAD < _CPW)
        def _issue():
            nb = lax.rem(i + _LOOKAHEAD, _NBUF)

            @pl.when(i + _LOOKAHEAD >= _NBUF)
            def _drain():
                pg = w * _CPW + (i + _LOOKAHEAD - _NBUF)
                pltpu.make_async_copy(
                    bufs.at[nb],
                    out_hbm.at[pl.ds(pg * _CHUNK, _CHUNK)],
                    wsem.at[nb]).wait()

            pltpu.async_copy(
                table_hbm.at[idx_all.at[i + _LOOKAHEAD]],
                bufs.at[nb], gsem.at[nb])

        return carry

    lax.fori_loop(0, _CPW, chunk_body, 0)

    for j in range(_NBUF):
        i = _CPW - _NBUF + j
        b = i % _NBUF
        g = w * _CPW + i
        pltpu.make_async_copy(
            bufs.at[b], out_hbm.at[pl.ds(g * _CHUNK, _CHUNK)],
            wsem.at[b]).wait()


@jax.jit
def _sc_call(ids2, item_table, pos_table, ln_gamma, ln_beta):
    mesh = plsc.VectorSubcoreMesh(
        core_axis_name="c", subcore_axis_name="s",
        num_cores=_NC, num_subcores=_NS)
    fn = pl.kernel(
        _sc_body,
        out_type=jax.ShapeDtypeStruct((_B * _L, _H), jnp.float32),
        mesh=mesh,
        compiler_params=pltpu.CompilerParams(
            needs_layout_passes=False, use_tc_tiling_on_sc=False),
        scratch_types=[
            pltpu.VMEM((_CPW, _CHUNK), jnp.int32),
            pltpu.VMEM((_L, _H), jnp.float32),
            pltpu.VMEM((2, _H), jnp.float32),
            pltpu.VMEM((_NBUF, _CHUNK, _H), jnp.float32),
            pltpu.SemaphoreType.DMA((_NBUF,)),
            pltpu.SemaphoreType.DMA((_NBUF,)),
        ],
    )
    return fn(ids2, item_table, pos_table, ln_gamma, ln_beta)


def kernel(input_ids, item_table, pos_table, ln_gamma, ln_beta):
    ids2 = input_ids.reshape(_B * _L // _CHUNK, _CHUNK)
    out = _sc_call(ids2, item_table, pos_table, ln_gamma, ln_beta)
    return out.reshape(_B, _L, _H)


# speedup vs baseline: 1.0607x; 1.0607x over previous
import functools

import jax
import jax.numpy as jnp
from jax import lax
from jax.experimental import pallas as pl
from jax.experimental.pallas import tpu as pltpu
from jax.experimental.pallas import tpu_sc as plsc

_B = 4096
_L = 200
_H = 64
_LANES = 16
_NC = 2
_NS = 16
_NW = _NC * _NS

_CHUNK = 128
_CPW = (_B * _L) // (_CHUNK * _NW)
_NBUF = 4
_LOOKAHEAD = 2
_EPS = 1e-12


def _rsqrt_vec(x):
    i = plsc.bitcast(x, jnp.int32)
    i = jnp.int32(0x5F3759DF) - lax.shift_right_arithmetic(i, 1)
    y = plsc.bitcast(i, jnp.float32)
    for _ in range(2):
        y = y * (1.5 - 0.5 * x * y * y)
    return y


def _sc_body(ids_hbm, table_hbm, pos_hbm, gamma_hbm, beta_hbm, out_hbm,
             idx_all, pos_v, gb_v, bufs, gsem, wsem):
    w = lax.axis_index("s") * _NC + lax.axis_index("c")

    pltpu.sync_copy(ids_hbm.at[pl.ds(w * _CPW, _CPW)], idx_all)
    pltpu.sync_copy(pos_hbm, pos_v)
    pltpu.sync_copy(gamma_hbm, gb_v.at[0])
    pltpu.sync_copy(beta_hbm, gb_v.at[1])

    gvecs = [gb_v[0, pl.ds(k * _LANES, _LANES)] for k in range(_H // _LANES)]
    bvecs = [gb_v[1, pl.ds(k * _LANES, _LANES)] for k in range(_H // _LANES)]

    iot = lax.iota(jnp.int32, _LANES)
    perms = [iot ^ d for d in (1, 2, 4, 8)]

    def allsum(x):
        for p in perms:
            x = x + x.at[p].get(mode="promise_in_bounds")
        return x

    for j in range(_LOOKAHEAD):
        pltpu.async_copy(table_hbm.at[idx_all.at[j]], bufs.at[j], gsem.at[j])

    def chunk_body(i, carry):
        b = lax.rem(i, _NBUF)
        g = w * _CPW + i
        pltpu.make_async_copy(
            table_hbm.at[idx_all.at[i]], bufs.at[b], gsem.at[b]).wait()

        lbase = lax.rem(g * _CHUNK, _L)

        def row_body(it, c):
            for u in range(2):
                r = it * 2 + u
                l0 = lbase + r
                l = jnp.where(l0 >= _L, l0 - _L, l0)
                xs = []
                for k in range(_H // _LANES):
                    xs.append(bufs[b, r, pl.ds(k * _LANES, _LANES)]
                              + pos_v[l, pl.ds(k * _LANES, _LANES)])
                s = (xs[0] + xs[1]) + (xs[2] + xs[3])
                sq = ((xs[0] * xs[0] + xs[1] * xs[1])
                      + (xs[2] * xs[2] + xs[3] * xs[3]))
                mean_v = allsum(s) * (1.0 / _H)
                ex2_v = allsum(sq) * (1.0 / _H)
                var_v = jnp.maximum(ex2_v - mean_v * mean_v, 0.0)
                rstd = _rsqrt_vec(var_v + _EPS)
                for k in range(_H // _LANES):
                    y = (xs[k] - mean_v) * (rstd * gvecs[k]) + bvecs[k]
                    bufs[b, r, pl.ds(k * _LANES, _LANES)] = y
            return c

        lax.fori_loop(0, _CHUNK // 2, row_body, 0)

        pltpu.async_copy(
            bufs.at[b], out_hbm.at[pl.ds(g * _CHUNK, _CHUNK)], wsem.at[b])

        @pl.when(i + _LOOKAHEAD < _CPW)
        def _issue():
            nb = lax.rem(i + _LOOKAHEAD, _NBUF)

            @pl.when(i + _LOOKAHEAD >= _NBUF)
            def _drain():
                pg = w * _CPW + (i + _LOOKAHEAD - _NBUF)
                pltpu.make_async_copy(
                    bufs.at[nb],
                    out_hbm.at[pl.ds(pg * _CHUNK, _CHUNK)],
                    wsem.at[nb]).wait()

            pltpu.async_copy(
                table_hbm.at[idx_all.at[i + _LOOKAHEAD]],
                bufs.at[nb], gsem.at[nb])

        return carry

    lax.fori_loop(0, _CPW, chunk_body, 0)

    for j in range(_NBUF):
        i = _CPW - _NBUF + j
        b = i % _NBUF
        g = w * _CPW + i
        pltpu.make_async_copy(
            bufs.at[b], out_hbm.at[pl.ds(g * _CHUNK, _CHUNK)],
            wsem.at[b]).wait()


@jax.jit
def _sc_call(ids2, item_table, pos_table, ln_gamma, ln_beta):
    mesh = plsc.VectorSubcoreMesh(
        core_axis_name="c", subcore_axis_name="s",
        num_cores=_NC, num_subcores=_NS)
    fn = pl.kernel(
        _sc_body,
        out_type=jax.ShapeDtypeStruct((_B * _L, _H), jnp.float32),
        mesh=mesh,
        compiler_params=pltpu.CompilerParams(
            needs_layout_passes=False, use_tc_tiling_on_sc=False),
        scratch_types=[
            pltpu.VMEM((_CPW, _CHUNK), jnp.int32),
            pltpu.VMEM((_L, _H), jnp.float32),
            pltpu.VMEM((2, _H), jnp.float32),
            pltpu.VMEM((_NBUF, _CHUNK, _H), jnp.float32),
            pltpu.SemaphoreType.DMA((_NBUF,)),
            pltpu.SemaphoreType.DMA((_NBUF,)),
        ],
    )
    return fn(ids2, item_table, pos_table, ln_gamma, ln_beta)


def kernel(input_ids, item_table, pos_table, ln_gamma, ln_beta):
    ids2 = input_ids.reshape(_B * _L // _CHUNK, _CHUNK)
    out = _sc_call(ids2, item_table, pos_table, ln_gamma, ln_beta)
    return out.reshape(_B, _L, _H)
